# Initial kernel scaffold; baseline (speedup 1.0000x reference)
import jax
import jax.numpy as jnp
from jax.experimental import pallas as pl


def kernel(h):
    raise NotImplementedError("write your pallas kernel here")


# speedup vs baseline: 1.7296x; 1.7296x over previous
import functools

import jax
import jax.numpy as jnp
from jax import lax
from jax.experimental import pallas as pl
from jax.experimental.pallas import tpu as pltpu
from jax.experimental.pallas import tpu_sc as plsc

_CHANNELS = (3, 17, 42, 77, 101, 200, 333, 450)
_N = 100000
_D = 512
_NW = 32
_CHUNK = 80
_NCHUNKS = _N // _CHUNK
_PAIRS = _CHUNK // 2

_mesh = plsc.VectorSubcoreMesh(core_axis_name="c", subcore_axis_name="s")


@functools.partial(
    pl.kernel,
    mesh=_mesh,
    compiler_params=pltpu.CompilerParams(needs_layout_passes=False),
    out_type=jax.ShapeDtypeStruct((_N, _D), jnp.float32),
    scratch_types=[
        pltpu.VMEM((_CHUNK, _D), jnp.float32),
        pltpu.SemaphoreType.DMA,
    ],
)
def _zero_channels_sc(h_hbm, out_hbm, buf, sem):
    wid = lax.axis_index("s") * 2 + lax.axis_index("c")

    lane = lax.iota(jnp.int32, 16)
    half = lane // 8
    lane8 = lane % 8
    cols = jnp.int32(0)
    for i, ch in enumerate(_CHANNELS):
        cols = jnp.where(lane8 == i, jnp.int32(ch), cols)
    zeros = (lane * 0).astype(jnp.float32)

    nmine = jnp.where(wid < _NCHUNKS % _NW, _NCHUNKS // _NW + 1, _NCHUNKS // _NW)

    def chunk_body(i, carry):
        r0 = (wid + i * _NW) * _CHUNK
        pltpu.sync_copy(h_hbm.at[pl.ds(r0, _CHUNK)], buf)
        for j in range(_PAIRS):
            rows = half + (2 * j)
            plsc.store_scatter(buf, [rows, cols], zeros)
        pltpu.sync_copy(buf, out_hbm.at[pl.ds(r0, _CHUNK)])
        return carry

    lax.fori_loop(0, nmine, chunk_body, 0)


def kernel(h):
    return _zero_channels_sc(h)


# speedup vs baseline: 2.0347x; 1.1764x over previous
import functools

import jax
import jax.numpy as jnp
from jax import lax
from jax.experimental import pallas as pl
from jax.experimental.pallas import tpu as pltpu
from jax.experimental.pallas import tpu_sc as plsc

_CHANNELS = (3, 17, 42, 77, 101, 200, 333, 450)
_N = 100000
_D = 512
_NW = 32
_CHUNK = 80
_NCHUNKS = _N // _CHUNK
_PAIRS = _CHUNK // 2
_NMAX = -(-_NCHUNKS // _NW)

_mesh = plsc.VectorSubcoreMesh(core_axis_name="c", subcore_axis_name="s")


@functools.partial(
    pl.kernel,
    mesh=_mesh,
    compiler_params=pltpu.CompilerParams(needs_layout_passes=False),
    out_type=jax.ShapeDtypeStruct((_N, _D), jnp.float32),
    scratch_types=[
        pltpu.VMEM((2, _CHUNK, _D), jnp.float32),
        pltpu.SemaphoreType.DMA,
        pltpu.SemaphoreType.DMA,
        pltpu.SemaphoreType.DMA,
        pltpu.SemaphoreType.DMA,
    ],
)
def _zero_channels_sc(h_hbm, out_hbm, buf, in_s0, in_s1, out_s0, out_s1):
    wid = lax.axis_index("s") * 2 + lax.axis_index("c")
    in_sems = (in_s0, in_s1)
    out_sems = (out_s0, out_s1)

    lane = lax.iota(jnp.int32, 16)
    half = lane // 8
    lane8 = lane % 8
    cols = jnp.int32(0)
    for i, ch in enumerate(_CHANNELS):
        cols = jnp.where(lane8 == i, jnp.int32(ch), cols)
    zeros = (lane * 0).astype(jnp.float32)

    def _in_desc(i):
        b = i % 2
        r0 = (wid + i * _NW) * _CHUNK
        return pltpu.make_async_copy(
            h_hbm.at[pl.ds(r0, _CHUNK)], buf.at[b], in_sems[b]
        )

    def _out_desc(i):
        b = i % 2
        r0 = (wid + i * _NW) * _CHUNK
        return pltpu.make_async_copy(
            buf.at[b], out_hbm.at[pl.ds(r0, _CHUNK)], out_sems[b]
        )

    def copy_in(i):
        _in_desc(i).start()

    def copy_out(i):
        _out_desc(i).start()

    def process(i):
        _in_desc(i).wait()

        def pair(j, carry):
            plsc.store_scatter(buf.at[i % 2], [half + 2 * j, cols], zeros)
            return carry

        lax.fori_loop(0, _PAIRS, pair, 0)
        copy_out(i)

    last_valid = wid + (_NMAX - 1) * _NW < _NCHUNKS

    copy_in(0)
    for i in range(_NMAX):
        if i + 1 < _NMAX:
            if i >= 1:
                _out_desc(i - 1).wait()
            if i + 1 == _NMAX - 1:
                def start_last(i=i):
                    copy_in(i + 1)
                pl.when(last_valid)(start_last)
            else:
                copy_in(i + 1)
        if i == _NMAX - 1:
            pl.when(last_valid)(lambda i=i: process(i))
        else:
            process(i)

    _out_desc(_NMAX - 2).wait()

    def drain_last():
        _out_desc(_NMAX - 1).wait()

    pl.when(last_valid)(drain_last)


def kernel(h):
    return _zero_channels_sc(h)


# speedup vs baseline: 2.1848x; 1.0737x over previous
import functools

import jax
import jax.numpy as jnp
from jax import lax
from jax.experimental import pallas as pl
from jax.experimental.pallas import tpu as pltpu
from jax.experimental.pallas import tpu_sc as plsc

_CHANNELS = (3, 17, 42, 77, 101, 200, 333, 450)
_N = 100000
_D = 512
_NW = 32
_CHUNK = 80
_NCHUNKS = _N // _CHUNK
_PAIRS = _CHUNK // 2
_NMAX = -(-_NCHUNKS // _NW)
_NBUF = 3

_mesh = plsc.VectorSubcoreMesh(core_axis_name="c", subcore_axis_name="s")


@functools.partial(
    pl.kernel,
    mesh=_mesh,
    compiler_params=pltpu.CompilerParams(needs_layout_passes=False),
    out_type=jax.ShapeDtypeStruct((_N, _D), jnp.float32),
    scratch_types=[
        pltpu.VMEM((_NBUF, _CHUNK, _D), jnp.float32),
        pltpu.SemaphoreType.DMA,
        pltpu.SemaphoreType.DMA,
        pltpu.SemaphoreType.DMA,
        pltpu.SemaphoreType.DMA,
        pltpu.SemaphoreType.DMA,
        pltpu.SemaphoreType.DMA,
    ],
)
def _zero_channels_sc(h_hbm, out_hbm, buf, i_s0, i_s1, i_s2, o_s0, o_s1, o_s2):
    wid = lax.axis_index("s") * 2 + lax.axis_index("c")
    in_sems = (i_s0, i_s1, i_s2)
    out_sems = (o_s0, o_s1, o_s2)

    lane = lax.iota(jnp.int32, 16)
    half = lane // 8
    lane8 = lane % 8
    cols = jnp.int32(0)
    for i, ch in enumerate(_CHANNELS):
        cols = jnp.where(lane8 == i, jnp.int32(ch), cols)
    zeros = (lane * 0).astype(jnp.float32)

    def _in_desc(i):
        b = i % _NBUF
        r0 = (wid + i * _NW) * _CHUNK
        return pltpu.make_async_copy(
            h_hbm.at[pl.ds(r0, _CHUNK)], buf.at[b], in_sems[b]
        )

    def _out_desc(i):
        b = i % _NBUF
        r0 = (wid + i * _NW) * _CHUNK
        return pltpu.make_async_copy(
            buf.at[b], out_hbm.at[pl.ds(r0, _CHUNK)], out_sems[b]
        )

    def process(i):
        _in_desc(i).wait()

        def pair(j, carry):
            plsc.store_scatter(buf.at[i % _NBUF], [half + 2 * j, cols], zeros)
            return carry

        lax.fori_loop(0, _PAIRS, pair, 0)
        _out_desc(i).start()

    last_valid = wid + (_NMAX - 1) * _NW < _NCHUNKS

    _in_desc(0).start()
    _in_desc(1).start()
    waited_out = 0
    for i in range(_NMAX):
        if i + 2 < _NMAX:
            if i >= 1:
                _out_desc(i - 1).wait()
                waited_out = i
            _in_desc(i + 2).start()
        elif i + 2 == _NMAX:
            if i >= 1:
                _out_desc(i - 1).wait()
                waited_out = i

            def start_last(i=i):
                _in_desc(i + 2).start()

            pl.when(last_valid)(start_last)
        if i == _NMAX - 1:
            pl.when(last_valid)(lambda i=i: process(i))
        else:
            process(i)

    for i in range(waited_out, _NMAX - 1):
        _out_desc(i).wait()

    def drain_last():
        _out_desc(_NMAX - 1).wait()

    pl.when(last_valid)(drain_last)


def kernel(h):
    return _zero_channels_sc(h)
